# baseline (speedup 1.0000x reference)
import jax
import jax.numpy as jnp
from jax.experimental import pallas as pl
from jax.experimental.pallas import tpu as pltpu
from jax.experimental.pallas import tpu_sc as plsc

_RAD = 18
_EMB = 256
_H = 384
_W = 384
_N = _H * _W
_NBINS = 4096
_CHUNK = 16384
_NCHUNK = _N // _CHUNK
_NSLICE = 96
_NWORKER = 32
_K_RANK = 117964


def _to_key(x):
    b = jax.lax.bitcast_convert_type(x, jnp.int32)
    return b ^ ((b >> 31) & jnp.int32(0x7FFFFFFF))


def _from_key(k):
    b = k ^ ((k >> 31) & jnp.int32(0x7FFFFFFF))
    return jax.lax.bitcast_convert_type(b, jnp.float32)


def _mid(lo, hi):
    return (lo & hi) + ((lo ^ hi) >> 1)


_SB = 4


def _rsum(x):
    a = x[:, :, 0:128] + x[:, :, 128:256] + x[:, :, 256:384]
    a = a[:, 0:192] + a[:, 192:384]
    a = a[:, 0:96] + a[:, 96:192]
    a = a[:, 0:48] + a[:, 48:96]
    a = a[:, 0:24] + a[:, 24:48]
    a = a[:, 0:8] + a[:, 8:16] + a[:, 16:24]
    return jnp.sum(a, axis=(1, 2), keepdims=True)


def _rmin(x):
    a = jnp.minimum(jnp.minimum(x[:, :, 0:128], x[:, :, 128:256]), x[:, :, 256:384])
    a = jnp.minimum(a[:, 0:192], a[:, 192:384])
    a = jnp.minimum(a[:, 0:96], a[:, 96:192])
    a = jnp.minimum(a[:, 0:48], a[:, 48:96])
    a = jnp.minimum(a[:, 0:24], a[:, 24:48])
    a = jnp.minimum(jnp.minimum(a[:, 0:8], a[:, 8:16]), a[:, 16:24])
    return jnp.min(a, axis=(1, 2), keepdims=True)


def _rmax(x):
    a = jnp.maximum(jnp.maximum(x[:, :, 0:128], x[:, :, 128:256]), x[:, :, 256:384])
    a = jnp.maximum(a[:, 0:192], a[:, 192:384])
    a = jnp.maximum(a[:, 0:96], a[:, 96:192])
    a = jnp.maximum(a[:, 0:48], a[:, 48:96])
    a = jnp.maximum(a[:, 0:24], a[:, 24:48])
    a = jnp.maximum(jnp.maximum(a[:, 0:8], a[:, 8:16]), a[:, 16:24])
    return jnp.max(a, axis=(1, 2), keepdims=True)


def _count_le(key, pivot):
    return _rsum(jnp.where(key <= pivot, 1.0, 0.0))


def _step1(key, lo, hi, target):
    mid = _mid(lo, hi)
    pred = _count_le(key, mid) >= target
    return jnp.where(pred, lo, mid + 1), jnp.where(pred, mid, hi)


def _select_rank(key, rank_plus1, lo0, hi0, steps):
    def body(_, c):
        lo, hi = c
        return _step1(key, lo, hi, rank_plus1)

    lo, hi = jax.lax.fori_loop(0, steps, body, (lo0, hi0))
    return lo


def _sc_hist_body(vol_hbm, out_hbm, hist_v, buf_v, pref_v):
    cid = jax.lax.axis_index("c")
    sid = jax.lax.axis_index("s")
    wid = sid * 2 + cid
    lane_base = jax.lax.iota(jnp.int32, 16) * _NBINS
    ones = jnp.ones((16,), jnp.float32)

    def per_slice(r, _):
        sl = wid + _NWORKER * r
        zeros16 = jnp.zeros((16,), jnp.float32)

        def zero(i, _z):
            for j in range(8):
                hist_v[pl.ds(i * 128 + j * 16, 16)] = zeros16
            return 0

        jax.lax.fori_loop(0, (_NBINS * 16) // 128, zero, 0)

        def chunk(ci, _c):
            pltpu.sync_copy(
                vol_hbm.at[pl.ds(sl * _N + ci * _CHUNK, _CHUNK)], buf_v)

            def vec(vi, _v):
                for j in range(8):
                    x = buf_v[pl.ds(vi * 128 + j * 16, 16)]
                    b = jax.lax.bitcast_convert_type(x, jnp.int32)
                    key = b ^ ((b >> 31) & jnp.int32(0x7FFFFFFF))
                    bin_ = (key >> 20) + jnp.int32(2048)
                    plsc.addupdate_scatter(hist_v, [lane_base + bin_], ones)
                return 0

            jax.lax.fori_loop(0, _CHUNK // 128, vec, 0)
            return 0

        jax.lax.fori_loop(0, _NCHUNK, chunk, 0)

        def pref(j, running):
            acc = hist_v[pl.ds(j * 16, 16)]
            for l in range(1, 16):
                acc = acc + hist_v[pl.ds(l * _NBINS + j * 16, 16)]
            cs = plsc.cumsum(acc) + running
            pref_v[pl.ds(j * 16, 16)] = cs
            return jnp.max(cs, axis=0)

        jax.lax.fori_loop(0, _NBINS // 16, pref, jnp.float32(0.0))
        pltpu.sync_copy(pref_v, out_hbm.at[pl.ds(sl * _NBINS, _NBINS)])
        return 0

    jax.lax.fori_loop(0, _NSLICE // _NWORKER, per_slice, 0)


def _sc_prefix(volume_flat):
    mesh = plsc.VectorSubcoreMesh(core_axis_name="c", subcore_axis_name="s")
    fn = pl.kernel(
        _sc_hist_body,
        mesh=mesh,
        out_type=jax.ShapeDtypeStruct((_NSLICE * _NBINS,), jnp.float32),
        scratch_types=[
            pltpu.VMEM((_NBINS * 16,), jnp.float32),
            pltpu.VMEM((_CHUNK,), jnp.float32),
            pltpu.VMEM((_NBINS,), jnp.float32),
        ],
        compiler_params=pltpu.CompilerParams(needs_layout_passes=False),
    )
    return fn(volume_flat)


def _bucket_bounds(prow, target):
    t2 = target.reshape(_SB, 1)
    b = jnp.sum(jnp.where(prow < t2, 1.0, 0.0), axis=1, keepdims=True)
    b = b.reshape(_SB, 1, 1).astype(jnp.int32)
    lo = (b - jnp.int32(2048)) << 20
    return lo, lo + jnp.int32(0xFFFFF)


def _slice_kernel(vol_ref, pref_ref, wt_ref, bp_ref, g_ref, bb_ref, out_ref):
    sl = vol_ref[...]
    prow = pref_ref[0]
    key = _to_key(sl)

    ttarget = jnp.full((_SB, 1, 1), _K_RANK + 1, jnp.float32)
    tlo, thi = _bucket_bounds(prow, ttarget)
    tkey = _select_rank(key, ttarget, tlo, thi, 20)
    thr = _from_key(tkey)

    maskb = sl >= thr
    m = maskb.astype(jnp.float32)
    count = _rsum(m)

    mean = _rsum(sl * m) / count
    d = sl - mean
    d2 = d * d
    var = _rsum(d2 * m) / count
    std = jnp.sqrt(var)
    std_eps = jnp.maximum(std, 1e-6)
    m3 = _rsum(d2 * d * m) / count
    m4 = _rsum(d2 * d2 * m) / count
    skew = jnp.clip(m3 / (std_eps * std_eps * std_eps), -50.0, 50.0)
    kurt = jnp.clip(m4 / (std_eps * std_eps * std_eps * std_eps), 0.0, 100.0)

    big = jnp.float32(jnp.inf)
    vmin = _rmin(jnp.where(maskb, sl, big))
    vmax = _rmax(jnp.where(maskb, sl, -big))
    sqmean = _rsum(sl * sl * m) / count
    absmean = _rsum(jnp.abs(sl) * m) / count

    count_i = count.astype(jnp.int32)
    n_minus_count = jnp.int32(_N) - count_i

    ranks = []
    fracs = []
    for q in (0.25, 0.5, 0.75):
        pos = jnp.float32(q) * (count - 1.0)
        lo_i = jnp.floor(pos).astype(jnp.int32)
        fracs.append(pos - lo_i.astype(jnp.float32))
        ranks.append(n_minus_count + lo_i)

    targets = [ranks[j].astype(jnp.float32) + 1.0 for j in range(3)]
    b1 = _bucket_bounds(prow, targets[0])
    b2 = _bucket_bounds(prow, targets[1])
    b3 = _bucket_bounds(prow, targets[2])

    def qbody(_, c):
        l1, h1, l2, h2, l3, h3 = c
        l1, h1 = _step1(key, l1, h1, targets[0])
        l2, h2 = _step1(key, l2, h2, targets[1])
        l3, h3 = _step1(key, l3, h3, targets[2])
        return (l1, h1, l2, h2, l3, h3)

    l1, _, l2, _, l3, _ = jax.lax.fori_loop(
        0, 20, qbody, (b1[0], b1[1], b2[0], b2[1], b3[0], b3[1]))
    klo = (l1, l2, l3)

    imax = jnp.int32(0x7FFFFFFF)
    qvals = []
    for j in range(3):
        kj = klo[j]
        cle = _count_le(key, kj)
        nxt = _rmin(jnp.where(key > kj, key, imax))
        khi = jnp.where(cle >= ranks[j].astype(jnp.float32) + 2.0, kj, nxt)
        vlo = _from_key(kj)
        vhi = _from_key(khi)
        f = fracs[j]
        qvals.append(jnp.where(f > 0.0, vlo * (1.0 - f) + vhi * f, vlo))
    q25, q50, q75 = qvals

    rng = vmax - vmin
    safe_rng = jnp.where(rng > 0, rng, 1.0)
    idx = jnp.clip(jnp.floor((sl - vmin) / safe_rng * 16.0).astype(jnp.int32), 0, 15)
    hist = [_rsum(jnp.where(idx == k, m, 0.0)) for k in range(16)]
    hsum = hist[0]
    for k in range(1, 16):
        hsum = hsum + hist[k]
    hden = jnp.maximum(hsum, 1.0)
    ent = jnp.float32(0.0)
    for k in range(16):
        p = jnp.maximum(hist[k] / hden, 1e-6)
        ent = ent - p * jnp.log(p)
    ent = jnp.where(jnp.abs(vmin - vmax) <= 1e-8 + 1e-5 * jnp.abs(vmax), 0.0, ent)

    gy = jnp.concatenate(
        [sl[:, 1:2] - sl[:, 0:1], (sl[:, 2:] - sl[:, :-2]) * 0.5,
         sl[:, _H - 1:_H] - sl[:, _H - 2:_H - 1]],
        axis=1)
    gx = jnp.concatenate(
        [sl[:, :, 1:2] - sl[:, :, 0:1], (sl[:, :, 2:] - sl[:, :, :-2]) * 0.5,
         sl[:, :, _W - 1:_W] - sl[:, :, _W - 2:_W - 1]],
        axis=2)
    gm = jnp.sqrt(gy * gy + gx * gx)
    gm_mean = _rsum(gm * m) / count
    gd = gm - gm_mean
    gm_std = jnp.sqrt(_rsum(gd * gd * m) / count)

    rows = jax.lax.broadcasted_iota(jnp.int32, (_SB, _H, _W), 1).astype(jnp.float32)
    cols = jax.lax.broadcasted_iota(jnp.int32, (_SB, _H, _W), 2).astype(jnp.float32)
    center_y = _rsum(rows * m) / count / jnp.float32(_H - 1)
    center_x = _rsum(cols * m) / count / jnp.float32(_W - 1)
    frac_mask = count / jnp.float32(_N)

    feats = jnp.concatenate([
        s.reshape(_SB, 1) for s in (
            mean, std, vmin, vmax, q25, q50, q75, sqmean, ent, skew, kurt,
            frac_mask, gm_mean, gm_std, center_y, center_x, frac_mask,
            absmean)
    ], axis=1)

    x = jnp.dot(feats, wt_ref[...], preferred_element_type=jnp.float32)
    x = x + bp_ref[...]
    mu = jnp.mean(x, axis=1, keepdims=True)
    xc = x - mu
    v = jnp.mean(xc * xc, axis=1, keepdims=True)
    xhat = xc * jax.lax.rsqrt(v + 1e-5)
    out_ref[0] = xhat * g_ref[...] + bb_ref[...]


@jax.jit
def _run(volume, W_proj, b_proj, ln_g, ln_b):
    B, D, H, Wd = volume.shape
    slices = volume.reshape(B * D, H, Wd)
    ngrid = B * D // _SB
    prefix = _sc_prefix(volume.reshape(-1)).reshape(ngrid, _SB, _NBINS)
    wt = W_proj.T
    tokens = pl.pallas_call(
        _slice_kernel,
        grid=(ngrid,),
        in_specs=[
            pl.BlockSpec((_SB, H, Wd), lambda i: (i, 0, 0)),
            pl.BlockSpec((1, _SB, _NBINS), lambda i: (i, 0, 0)),
            pl.BlockSpec((_RAD, _EMB), lambda i: (0, 0)),
            pl.BlockSpec((1, _EMB), lambda i: (0, 0)),
            pl.BlockSpec((1, _EMB), lambda i: (0, 0)),
            pl.BlockSpec((1, _EMB), lambda i: (0, 0)),
        ],
        out_specs=pl.BlockSpec((1, _SB, _EMB), lambda i: (i, 0, 0)),
        out_shape=jax.ShapeDtypeStruct((ngrid, _SB, _EMB), jnp.float32),
        compiler_params=pltpu.CompilerParams(
            dimension_semantics=("arbitrary",)),
    )(slices, prefix, wt, b_proj.reshape(1, _EMB), ln_g.reshape(1, _EMB),
      ln_b.reshape(1, _EMB))
    tokens = tokens.reshape(B, D, _EMB)
    padding_mask = jnp.zeros((B, D), dtype=bool)
    return tokens, padding_mask


def kernel(volume, W_proj, b_proj, ln_g, ln_b):
    return _run(volume, W_proj, b_proj, ln_g, ln_b)


# speedup vs baseline: 1.0620x; 1.0620x over previous
import jax
import jax.numpy as jnp
from jax.experimental import pallas as pl
from jax.experimental.pallas import tpu as pltpu
from jax.experimental.pallas import tpu_sc as plsc

_RAD = 18
_EMB = 256
_H = 384
_W = 384
_N = _H * _W
_NBINS = 4096
_CHUNK = 16384
_NCHUNK = _N // _CHUNK
_NSLICE = 96
_NWORKER = 32
_K_RANK = 117964


def _to_key(x):
    b = jax.lax.bitcast_convert_type(x, jnp.int32)
    return b ^ ((b >> 31) & jnp.int32(0x7FFFFFFF))


def _from_key(k):
    b = k ^ ((k >> 31) & jnp.int32(0x7FFFFFFF))
    return jax.lax.bitcast_convert_type(b, jnp.float32)


def _mid(lo, hi):
    return (lo & hi) + ((lo ^ hi) >> 1)


_SB = 4


def _rsum(x):
    a = x[:, :, 0:128] + x[:, :, 128:256] + x[:, :, 256:384]
    a = a[:, 0:192] + a[:, 192:384]
    a = a[:, 0:96] + a[:, 96:192]
    a = a[:, 0:48] + a[:, 48:96]
    a = a[:, 0:24] + a[:, 24:48]
    a = a[:, 0:8] + a[:, 8:16] + a[:, 16:24]
    return jnp.sum(a, axis=(1, 2), keepdims=True)


def _rmin(x):
    a = jnp.minimum(jnp.minimum(x[:, :, 0:128], x[:, :, 128:256]), x[:, :, 256:384])
    a = jnp.minimum(a[:, 0:192], a[:, 192:384])
    a = jnp.minimum(a[:, 0:96], a[:, 96:192])
    a = jnp.minimum(a[:, 0:48], a[:, 48:96])
    a = jnp.minimum(a[:, 0:24], a[:, 24:48])
    a = jnp.minimum(jnp.minimum(a[:, 0:8], a[:, 8:16]), a[:, 16:24])
    return jnp.min(a, axis=(1, 2), keepdims=True)


def _rmax(x):
    a = jnp.maximum(jnp.maximum(x[:, :, 0:128], x[:, :, 128:256]), x[:, :, 256:384])
    a = jnp.maximum(a[:, 0:192], a[:, 192:384])
    a = jnp.maximum(a[:, 0:96], a[:, 96:192])
    a = jnp.maximum(a[:, 0:48], a[:, 48:96])
    a = jnp.maximum(a[:, 0:24], a[:, 24:48])
    a = jnp.maximum(jnp.maximum(a[:, 0:8], a[:, 8:16]), a[:, 16:24])
    return jnp.max(a, axis=(1, 2), keepdims=True)


def _count_le(key, pivot):
    return _rsum(jnp.where(key <= pivot, 1.0, 0.0))


def _step1(key, lo, hi, target):
    mid = _mid(lo, hi)
    pred = _count_le(key, mid) >= target
    return jnp.where(pred, lo, mid + 1), jnp.where(pred, mid, hi)


def _select_rank(key, rank_plus1, lo0, hi0, steps):
    def body(_, c):
        lo, hi = c
        return _step1(key, lo, hi, rank_plus1)

    lo, hi = jax.lax.fori_loop(0, steps, body, (lo0, hi0))
    return lo


def _sc_hist_body(vol_hbm, out_hbm, hist_v, buf_v, pref_v, sem_v):
    cid = jax.lax.axis_index("c")
    sid = jax.lax.axis_index("s")
    wid = sid * 2 + cid
    lane_base = jax.lax.iota(jnp.int32, 16) * _NBINS
    ones = jnp.ones((16,), jnp.float32)

    def per_slice(r, _):
        sl = wid + _NWORKER * r
        zeros16 = jnp.zeros((16,), jnp.float32)

        def zero(i, _z):
            for j in range(8):
                hist_v[pl.ds(i * 128 + j * 16, 16)] = zeros16
            return 0

        jax.lax.fori_loop(0, (_NBINS * 16) // 128, zero, 0)

        def _cp(c, bi):
            return pltpu.make_async_copy(
                vol_hbm.at[pl.ds(sl * _N + c * _CHUNK, _CHUNK)],
                buf_v.at[bi], sem_v.at[bi])

        def _process(bi):
            def vec(vi, _v):
                for j in range(8):
                    x = buf_v[bi, pl.ds(vi * 128 + j * 16, 16)]
                    b = jax.lax.bitcast_convert_type(x, jnp.int32)
                    key = b ^ ((b >> 31) & jnp.int32(0x7FFFFFFF))
                    bin_ = (key >> 20) + jnp.int32(2048)
                    plsc.addupdate_scatter(hist_v, [lane_base + bin_], ones)
                return 0

            jax.lax.fori_loop(0, _CHUNK // 128, vec, 0)

        _cp(0, 0).start()
        for c in range(_NCHUNK):
            if c + 1 < _NCHUNK:
                _cp(c + 1, (c + 1) % 2).start()
            _cp(c, c % 2).wait()
            _process(c % 2)

        def pref(j, running):
            acc = hist_v[pl.ds(j * 16, 16)]
            for l in range(1, 16):
                acc = acc + hist_v[pl.ds(l * _NBINS + j * 16, 16)]
            cs = plsc.cumsum(acc) + running
            pref_v[pl.ds(j * 16, 16)] = cs
            return jnp.max(cs, axis=0)

        jax.lax.fori_loop(0, _NBINS // 16, pref, jnp.float32(0.0))
        pltpu.sync_copy(pref_v, out_hbm.at[pl.ds(sl * _NBINS, _NBINS)])
        return 0

    jax.lax.fori_loop(0, _NSLICE // _NWORKER, per_slice, 0)


def _sc_prefix(volume_flat):
    mesh = plsc.VectorSubcoreMesh(core_axis_name="c", subcore_axis_name="s")
    fn = pl.kernel(
        _sc_hist_body,
        mesh=mesh,
        out_type=jax.ShapeDtypeStruct((_NSLICE * _NBINS,), jnp.float32),
        scratch_types=[
            pltpu.VMEM((_NBINS * 16,), jnp.float32),
            pltpu.VMEM((2, _CHUNK), jnp.float32),
            pltpu.VMEM((_NBINS,), jnp.float32),
            pltpu.SemaphoreType.DMA((2,)),
        ],
        compiler_params=pltpu.CompilerParams(needs_layout_passes=False),
    )
    return fn(volume_flat)


def _bucket_bounds(prow, target):
    t2 = target.reshape(_SB, 1)
    b = jnp.sum(jnp.where(prow < t2, 1.0, 0.0), axis=1, keepdims=True)
    b = b.reshape(_SB, 1, 1).astype(jnp.int32)
    lo = (b - jnp.int32(2048)) << 20
    return lo, lo + jnp.int32(0xFFFFF)


def _slice_kernel(vol_ref, pref_ref, wt_ref, bp_ref, g_ref, bb_ref, out_ref):
    sl = vol_ref[...]
    prow = pref_ref[0]
    key = _to_key(sl)

    ttarget = jnp.full((_SB, 1, 1), _K_RANK + 1, jnp.float32)
    tlo, thi = _bucket_bounds(prow, ttarget)
    tkey = _select_rank(key, ttarget, tlo, thi, 20)
    thr = _from_key(tkey)

    maskb = sl >= thr
    m = maskb.astype(jnp.float32)
    count = _rsum(m)

    mean = _rsum(sl * m) / count
    d = sl - mean
    d2 = d * d
    var = _rsum(d2 * m) / count
    std = jnp.sqrt(var)
    std_eps = jnp.maximum(std, 1e-6)
    m3 = _rsum(d2 * d * m) / count
    m4 = _rsum(d2 * d2 * m) / count
    skew = jnp.clip(m3 / (std_eps * std_eps * std_eps), -50.0, 50.0)
    kurt = jnp.clip(m4 / (std_eps * std_eps * std_eps * std_eps), 0.0, 100.0)

    big = jnp.float32(jnp.inf)
    vmin = _rmin(jnp.where(maskb, sl, big))
    vmax = _rmax(jnp.where(maskb, sl, -big))
    sqmean = _rsum(sl * sl * m) / count
    absmean = _rsum(jnp.abs(sl) * m) / count

    count_i = count.astype(jnp.int32)
    n_minus_count = jnp.int32(_N) - count_i

    ranks = []
    fracs = []
    for q in (0.25, 0.5, 0.75):
        pos = jnp.float32(q) * (count - 1.0)
        lo_i = jnp.floor(pos).astype(jnp.int32)
        fracs.append(pos - lo_i.astype(jnp.float32))
        ranks.append(n_minus_count + lo_i)

    targets = [ranks[j].astype(jnp.float32) + 1.0 for j in range(3)]
    b1 = _bucket_bounds(prow, targets[0])
    b2 = _bucket_bounds(prow, targets[1])
    b3 = _bucket_bounds(prow, targets[2])

    def qbody(_, c):
        l1, h1, l2, h2, l3, h3 = c
        l1, h1 = _step1(key, l1, h1, targets[0])
        l2, h2 = _step1(key, l2, h2, targets[1])
        l3, h3 = _step1(key, l3, h3, targets[2])
        return (l1, h1, l2, h2, l3, h3)

    l1, _, l2, _, l3, _ = jax.lax.fori_loop(
        0, 20, qbody, (b1[0], b1[1], b2[0], b2[1], b3[0], b3[1]))
    klo = (l1, l2, l3)

    imax = jnp.int32(0x7FFFFFFF)
    qvals = []
    for j in range(3):
        kj = klo[j]
        cle = _count_le(key, kj)
        nxt = _rmin(jnp.where(key > kj, key, imax))
        khi = jnp.where(cle >= ranks[j].astype(jnp.float32) + 2.0, kj, nxt)
        vlo = _from_key(kj)
        vhi = _from_key(khi)
        f = fracs[j]
        qvals.append(jnp.where(f > 0.0, vlo * (1.0 - f) + vhi * f, vlo))
    q25, q50, q75 = qvals

    rng = vmax - vmin
    safe_rng = jnp.where(rng > 0, rng, 1.0)
    idx = jnp.clip(jnp.floor((sl - vmin) / safe_rng * 16.0).astype(jnp.int32), 0, 15)
    hist = [_rsum(jnp.where(idx == k, m, 0.0)) for k in range(16)]
    hsum = hist[0]
    for k in range(1, 16):
        hsum = hsum + hist[k]
    hden = jnp.maximum(hsum, 1.0)
    ent = jnp.float32(0.0)
    for k in range(16):
        p = jnp.maximum(hist[k] / hden, 1e-6)
        ent = ent - p * jnp.log(p)
    ent = jnp.where(jnp.abs(vmin - vmax) <= 1e-8 + 1e-5 * jnp.abs(vmax), 0.0, ent)

    gy = jnp.concatenate(
        [sl[:, 1:2] - sl[:, 0:1], (sl[:, 2:] - sl[:, :-2]) * 0.5,
         sl[:, _H - 1:_H] - sl[:, _H - 2:_H - 1]],
        axis=1)
    gx = jnp.concatenate(
        [sl[:, :, 1:2] - sl[:, :, 0:1], (sl[:, :, 2:] - sl[:, :, :-2]) * 0.5,
         sl[:, :, _W - 1:_W] - sl[:, :, _W - 2:_W - 1]],
        axis=2)
    gm = jnp.sqrt(gy * gy + gx * gx)
    gm_mean = _rsum(gm * m) / count
    gd = gm - gm_mean
    gm_std = jnp.sqrt(_rsum(gd * gd * m) / count)

    rows = jax.lax.broadcasted_iota(jnp.int32, (_SB, _H, _W), 1).astype(jnp.float32)
    cols = jax.lax.broadcasted_iota(jnp.int32, (_SB, _H, _W), 2).astype(jnp.float32)
    center_y = _rsum(rows * m) / count / jnp.float32(_H - 1)
    center_x = _rsum(cols * m) / count / jnp.float32(_W - 1)
    frac_mask = count / jnp.float32(_N)

    feats = jnp.concatenate([
        s.reshape(_SB, 1) for s in (
            mean, std, vmin, vmax, q25, q50, q75, sqmean, ent, skew, kurt,
            frac_mask, gm_mean, gm_std, center_y, center_x, frac_mask,
            absmean)
    ], axis=1)

    x = jnp.dot(feats, wt_ref[...], preferred_element_type=jnp.float32)
    x = x + bp_ref[...]
    mu = jnp.mean(x, axis=1, keepdims=True)
    xc = x - mu
    v = jnp.mean(xc * xc, axis=1, keepdims=True)
    xhat = xc * jax.lax.rsqrt(v + 1e-5)
    out_ref[0] = xhat * g_ref[...] + bb_ref[...]


@jax.jit
def _run(volume, W_proj, b_proj, ln_g, ln_b):
    B, D, H, Wd = volume.shape
    slices = volume.reshape(B * D, H, Wd)
    ngrid = B * D // _SB
    prefix = _sc_prefix(volume.reshape(-1)).reshape(ngrid, _SB, _NBINS)
    wt = W_proj.T
    tokens = pl.pallas_call(
        _slice_kernel,
        grid=(ngrid,),
        in_specs=[
            pl.BlockSpec((_SB, H, Wd), lambda i: (i, 0, 0)),
            pl.BlockSpec((1, _SB, _NBINS), lambda i: (i, 0, 0)),
            pl.BlockSpec((_RAD, _EMB), lambda i: (0, 0)),
            pl.BlockSpec((1, _EMB), lambda i: (0, 0)),
            pl.BlockSpec((1, _EMB), lambda i: (0, 0)),
            pl.BlockSpec((1, _EMB), lambda i: (0, 0)),
        ],
        out_specs=pl.BlockSpec((1, _SB, _EMB), lambda i: (i, 0, 0)),
        out_shape=jax.ShapeDtypeStruct((ngrid, _SB, _EMB), jnp.float32),
        compiler_params=pltpu.CompilerParams(
            dimension_semantics=("arbitrary",)),
    )(slices, prefix, wt, b_proj.reshape(1, _EMB), ln_g.reshape(1, _EMB),
      ln_b.reshape(1, _EMB))
    tokens = tokens.reshape(B, D, _EMB)
    padding_mask = jnp.zeros((B, D), dtype=bool)
    return tokens, padding_mask


def kernel(volume, W_proj, b_proj, ln_g, ln_b):
    return _run(volume, W_proj, b_proj, ln_g, ln_b)
